# baseline (speedup 1.0000x reference)
import functools

import jax
import jax.numpy as jnp
from jax import lax
from jax.experimental import pallas as pl
from jax.experimental.pallas import tpu as pltpu
from jax.experimental.pallas import tpu_sc as plsc

BSZ = 16384
FEA = 64
NC = 2
NS = 16
NW = NC * NS
B_PER_W = BSZ // NW
LANES = 16


def _sc_bias_kernel(uid_hbm, iid_hbm, utab_hbm, itab_hbm, out_hbm,
                    uidx_v, iidx_v, urows_v, irows_v, sem_u, sem_i):
    wid = lax.axis_index("s") * NC + lax.axis_index("c")
    base = wid * B_PER_W
    pltpu.sync_copy(uid_hbm.at[pl.ds(base, B_PER_W)], uidx_v)
    pltpu.sync_copy(iid_hbm.at[pl.ds(base, B_PER_W)], iidx_v)
    cp_u = pltpu.async_copy(utab_hbm.at[uidx_v], urows_v, sem_u)
    cp_i = pltpu.async_copy(itab_hbm.at[iidx_v], irows_v, sem_i)
    cp_u.wait()
    cp_i.wait()
    for j in range(B_PER_W // LANES):
        sl = pl.ds(j * LANES, LANES)
        urows_v[sl] = urows_v[sl] + irows_v[sl]
    pltpu.sync_copy(urows_v, out_hbm.at[pl.ds(base, B_PER_W)])


def _sc_bias(batch_uid, batch_iid, utab, itab):
    mesh = plsc.VectorSubcoreMesh(
        core_axis_name="c", subcore_axis_name="s",
        num_cores=NC, num_subcores=NS)
    return pl.kernel(
        _sc_bias_kernel,
        out_type=jax.ShapeDtypeStruct((BSZ,), jnp.float32),
        mesh=mesh,
        scratch_types=[
            pltpu.VMEM((B_PER_W,), jnp.int32),
            pltpu.VMEM((B_PER_W,), jnp.int32),
            pltpu.VMEM((B_PER_W,), jnp.float32),
            pltpu.VMEM((B_PER_W,), jnp.float32),
            pltpu.SemaphoreType.DMA,
            pltpu.SemaphoreType.DMA,
        ],
    )(batch_uid, batch_iid, utab, itab)


def _tc_dot_kernel(go_ref, u_ref, i_ref, b_ref, o_ref):
    dot = jnp.sum(u_ref[...] * i_ref[...], axis=1, keepdims=True)
    o_ref[...] = dot + b_ref[...] + go_ref[0]


def _tc_dot(batch_userFea, batch_itemFea, bias, globalOffset):
    blk = 4096
    grid = (BSZ // blk,)
    return pl.pallas_call(
        _tc_dot_kernel,
        grid=grid,
        in_specs=[
            pl.BlockSpec(memory_space=pltpu.SMEM),
            pl.BlockSpec((blk, FEA), lambda g: (g, 0)),
            pl.BlockSpec((blk, FEA), lambda g: (g, 0)),
            pl.BlockSpec((blk, 1), lambda g: (g, 0)),
        ],
        out_specs=pl.BlockSpec((blk, 1), lambda g: (g, 0)),
        out_shape=jax.ShapeDtypeStruct((BSZ, 1), jnp.float32),
    )(globalOffset, batch_userFea, batch_itemFea, bias)


def kernel(batch_userFea, batch_itemFea, batch_uid, batch_iid,
           globalOffset, uid_userOffset, iid_itemOffset):
    utab = uid_userOffset.reshape(-1)
    itab = iid_itemOffset.reshape(-1)
    uid = batch_uid.astype(jnp.int32)
    iid = batch_iid.astype(jnp.int32)
    bias = _sc_bias(uid, iid, utab, itab).reshape(BSZ, 1)
    return _tc_dot(batch_userFea, batch_itemFea, bias, globalOffset)
